# baseline (speedup 1.0000x reference)
import jax
import jax.numpy as jnp
from jax import lax
from jax.experimental import pallas as pl
from jax.experimental.pallas import tpu as pltpu
from jax.experimental.pallas import tpu_sc as plsc

MAX_LEN = 2048
MAX_MEM = 2048
VOCAB = 32000
EMB = 128
PAD = 0
HID = 64
N = MAX_LEN + MAX_MEM


_NC, _NS = 2, 16
_NW = _NC * _NS
_BPW = N // _NW


_L = 16


def _gather_body(table_hbm, idx_hbm, out_hbm, idx_v, rows_v, sem):
    wid = lax.axis_index("s") * _NC + lax.axis_index("c")
    base = wid * _BPW
    pltpu.sync_copy(idx_hbm.at[pl.ds(base, _BPW)], idx_v)
    pltpu.async_copy(table_hbm.at[idx_v], rows_v, sem).wait()
    pltpu.sync_copy(rows_v, out_hbm.at[pl.ds(base, _BPW)])


def _sc_gather(emb, tokens):
    mesh = plsc.VectorSubcoreMesh(
        core_axis_name="c", subcore_axis_name="s",
        num_cores=_NC, num_subcores=_NS)
    return pl.kernel(
        _gather_body,
        out_type=jax.ShapeDtypeStruct((N, EMB), jnp.float32),
        mesh=mesh,
        scratch_types=[
            pltpu.VMEM((_BPW,), jnp.int32),
            pltpu.VMEM((_BPW, EMB), jnp.float32),
            pltpu.SemaphoreType.DMA,
        ],
    )(emb, tokens)


_G = 4
_K = MAX_LEN * EMB
_KB = _K // _G


_NSPL = 2
_RSP = HID // _NSPL


def _fused_body(tok_ref, x_hbm, w1_hbm, w2_hbm, b1_ref, b2_ref,
                tt_ref, ts_ref, buf0, buf1, xbuf, w2buf, sem, xsem, w2sem):
    bufs = (buf0, buf1)

    def copies(k):
        b = bufs[k % 2]
        return [pltpu.make_async_copy(
            w1_hbm.at[pl.ds(s * _RSP, _RSP), pl.ds(k * _KB, _KB)],
            b.at[pl.ds(s * _RSP, _RSP), :],
            sem.at[k % 2, s]) for s in range(_NSPL)]

    xcopy = pltpu.make_async_copy(x_hbm, xbuf, xsem)
    w2copy = pltpu.make_async_copy(w2_hbm, w2buf, w2sem)
    for c in copies(0) + [xcopy, w2copy] + copies(1):
        c.start()

    tok = tok_ref[...]
    keep = _dedup_keep(tok)

    xcopy.wait()
    acc = jnp.zeros((2, HID), jnp.float32)
    for k in range(_G):
        if 1 <= k < _G - 1:
            for c in copies(k + 1):
                c.start()
        for c in copies(k):
            c.wait()
        acc = acc + lax.dot_general(
            xbuf[:, pl.ds(k * _KB, _KB)], bufs[k % 2][...],
            (((1,), (1,)), ((), ())),
            preferred_element_type=jnp.float32)
    h = jnp.maximum(acc + b1_ref[...], 0.0)
    w2copy.wait()
    z = lax.dot_general(
        h, w2buf[...], (((1,), (1,)), ((), ())),
        preferred_element_type=jnp.float32) + b2_ref[...]
    sc = jax.nn.sigmoid(z).reshape(_R, _C)

    skey2, sval2 = _topk(keep, tok, sc)
    tt_ref[...] = sval2[0:_R // 2, :]
    ts_ref[...] = skey2[0:_R // 2, :]


def _fused_call(tok2, x, w1, w2, b1, b2):
    return pl.pallas_call(
        _fused_body,
        in_specs=[
            pl.BlockSpec(memory_space=pltpu.MemorySpace.VMEM),
            pl.BlockSpec(memory_space=pltpu.MemorySpace.HBM),
            pl.BlockSpec(memory_space=pltpu.MemorySpace.HBM),
            pl.BlockSpec(memory_space=pltpu.MemorySpace.HBM),
            pl.BlockSpec(memory_space=pltpu.MemorySpace.VMEM),
            pl.BlockSpec(memory_space=pltpu.MemorySpace.VMEM),
        ],
        out_specs=(pl.BlockSpec(memory_space=pltpu.MemorySpace.VMEM),
                   pl.BlockSpec(memory_space=pltpu.MemorySpace.VMEM)),
        out_shape=(
            jax.ShapeDtypeStruct((_R // 2, _C), jnp.int32),
            jax.ShapeDtypeStruct((_R // 2, _C), jnp.float32),
        ),
        scratch_shapes=[
            pltpu.VMEM((HID, _KB), jnp.float32),
            pltpu.VMEM((HID, _KB), jnp.float32),
            pltpu.VMEM((2, _K), jnp.float32),
            pltpu.VMEM((MAX_LEN, HID), jnp.float32),
            pltpu.SemaphoreType.DMA((2, _NSPL)),
            pltpu.SemaphoreType.DMA,
            pltpu.SemaphoreType.DMA,
        ],
    )(tok2, x, w1, w2, b1, b2)


_R, _C = 8, 512


def _iotas():
    r = lax.broadcasted_iota(jnp.int32, (_R, _C), 0)
    c = lax.broadcasted_iota(jnp.int32, (_R, _C), 1)
    return r, c


def _roll(x, shift, axis):
    return pltpu.roll(x, shift % x.shape[axis], axis)


def _partner(x, j):
    if j < _C:
        lower = (_iotas()[1] & j) == 0
        return jnp.where(lower, _roll(x, -j, 1), _roll(x, j, 1))
    m = j // _C
    lower = (_iotas()[0] & m) == 0
    return jnp.where(lower, _roll(x, -m, 0), _roll(x, m, 0))


def _bitonic(key, val, descending, tiebreak=False):
    r, c = _iotas()
    k = 2
    while k <= _R * _C:
        j = k // 2
        while j >= 1:
            pk = _partner(key, j)
            pv = _partner(val, j)
            lower = ((c & j) == 0) if j < _C else ((r & (j // _C)) == 0)
            if k < _C:
                asc = (c & k) == 0
            elif k // _C < _R:
                asc = (r & (k // _C)) == 0
            else:
                asc = jnp.full((_R, _C), not descending)
            if k // _C < _R and descending:
                asc = jnp.logical_not(asc)
            keep_small = lower == asc
            if tiebreak:
                tie = pk == key
                p_lt = (pk < key) | (tie & (pv > val))
                p_gt = (pk > key) | (tie & (pv < val))
                swap = (keep_small & p_lt) | (~keep_small & p_gt)
            else:
                swap = (keep_small & (pk < key)) | (~keep_small & (pk > key))
            key = jnp.where(swap, pk, key)
            val = jnp.where(swap, pv, val)
            j //= 2
        k *= 2
    return key, val


def _bitonic_key(key, descending):
    r, c = _iotas()
    k = 2
    while k <= _R * _C:
        j = k // 2
        while j >= 1:
            pk = _partner(key, j)
            lower = ((c & j) == 0) if j < _C else ((r & (j // _C)) == 0)
            if k < _C:
                asc = (c & k) == 0
            elif k // _C < _R:
                asc = (r & (k // _C)) == 0
            else:
                asc = jnp.full((_R, _C), not descending)
            if k // _C < _R and descending:
                asc = jnp.logical_not(asc)
            keep_small = lower == asc
            swap = (keep_small & (pk < key)) | (~keep_small & (pk > key))
            key = jnp.where(swap, pk, key)
            j //= 2
        k *= 2
    return key


def _dedup_keep(tok):
    r, c = _iotas()
    pos = r * _C + c
    skey1 = _bitonic_key(tok * N + pos, descending=False)
    stok = skey1 >> 12
    nxt1 = _roll(stok, -1, 1)
    nxt = jnp.where(c == _C - 1, _roll(nxt1, -1, 0), nxt1)
    rep = (stok != nxt) | ((r == _R - 1) & (c == _C - 1))
    spos = skey1 & (N - 1)
    _, rep_pos = _bitonic(spos, rep.astype(jnp.int32), descending=False)
    return (rep_pos != 0) & (tok != PAD)


def _topk(keep, tok, sc):
    key2 = jnp.where(keep, sc, jnp.float32(-1e20))
    val2 = jnp.where(keep, tok, PAD)
    return _bitonic(key2, val2, descending=True, tiebreak=True)


def kernel(input_tokens, memory_context, emb, W1, b1, W2, b2):
    tokens = jnp.concatenate([input_tokens, memory_context], axis=0)
    rows = _sc_gather(emb, tokens)
    x = rows.reshape(2, _K)
    tt, ts = _fused_call(tokens.reshape(_R, _C), x, W1, W2,
                         b1.reshape(1, HID), b2.reshape(1, MAX_LEN))
    return tt.reshape(MAX_MEM), ts.reshape(MAX_MEM)


# speedup vs baseline: 1.0476x; 1.0476x over previous
import jax
import jax.numpy as jnp
from jax import lax
from jax.experimental import pallas as pl
from jax.experimental.pallas import tpu as pltpu
from jax.experimental.pallas import tpu_sc as plsc

MAX_LEN = 2048
MAX_MEM = 2048
VOCAB = 32000
EMB = 128
PAD = 0
HID = 64
N = MAX_LEN + MAX_MEM


_NC, _NS = 2, 16
_NW = _NC * _NS
_BPW = N // _NW


_L = 16


def _gather_body(table_hbm, idx_hbm, out_hbm, idx_v, rows_v, sem):
    wid = lax.axis_index("s") * _NC + lax.axis_index("c")
    base = wid * _BPW
    pltpu.sync_copy(idx_hbm.at[pl.ds(base, _BPW)], idx_v)
    pltpu.async_copy(table_hbm.at[idx_v], rows_v, sem).wait()
    pltpu.sync_copy(rows_v, out_hbm.at[pl.ds(base, _BPW)])


def _sc_gather(emb, tokens):
    mesh = plsc.VectorSubcoreMesh(
        core_axis_name="c", subcore_axis_name="s",
        num_cores=_NC, num_subcores=_NS)
    return pl.kernel(
        _gather_body,
        out_type=jax.ShapeDtypeStruct((N, EMB), jnp.float32),
        mesh=mesh,
        scratch_types=[
            pltpu.VMEM((_BPW,), jnp.int32),
            pltpu.VMEM((_BPW, EMB), jnp.float32),
            pltpu.SemaphoreType.DMA,
        ],
    )(emb, tokens)


_G = 4
_K = MAX_LEN * EMB
_KB = _K // _G


_NSPL = 2
_RSP = HID // _NSPL


def _fused_body(tok_ref, x_ref, w1_hbm, w2_ref, b1_ref, b2_ref,
                tt_ref, ts_ref, buf0, buf1, sem):
    bufs = (buf0, buf1)

    def copies(k):
        b = bufs[k % 2]
        return [pltpu.make_async_copy(
            w1_hbm.at[pl.ds(s * _RSP, _RSP), pl.ds(k * _KB, _KB)],
            b.at[pl.ds(s * _RSP, _RSP), :],
            sem.at[k % 2, s]) for s in range(_NSPL)]

    for c in copies(0) + copies(1):
        c.start()

    tok = tok_ref[...]
    keep = _dedup_keep(tok)

    acc = jnp.zeros((2, HID), jnp.float32)
    for k in range(_G):
        if 1 <= k < _G - 1:
            for c in copies(k + 1):
                c.start()
        for c in copies(k):
            c.wait()
        acc = acc + lax.dot_general(
            x_ref[:, pl.ds(k * _KB, _KB)], bufs[k % 2][...],
            (((1,), (1,)), ((), ())),
            preferred_element_type=jnp.float32)
    h = jnp.maximum(acc + b1_ref[...], 0.0)
    z = lax.dot_general(
        h, w2_ref[...], (((1,), (1,)), ((), ())),
        preferred_element_type=jnp.float32) + b2_ref[...]
    sc = jax.nn.sigmoid(z).reshape(_R, _C)

    skey2, sval2 = _topk(keep, tok, sc)
    tt_ref[...] = sval2[0:_R // 2, :]
    ts_ref[...] = skey2[0:_R // 2, :]


def _fused_call(tok2, x, w1, w2, b1, b2):
    return pl.pallas_call(
        _fused_body,
        in_specs=[
            pl.BlockSpec(memory_space=pltpu.MemorySpace.VMEM),
            pl.BlockSpec(memory_space=pltpu.MemorySpace.VMEM),
            pl.BlockSpec(memory_space=pltpu.MemorySpace.HBM),
            pl.BlockSpec(memory_space=pltpu.MemorySpace.VMEM),
            pl.BlockSpec(memory_space=pltpu.MemorySpace.VMEM),
            pl.BlockSpec(memory_space=pltpu.MemorySpace.VMEM),
        ],
        out_specs=(pl.BlockSpec(memory_space=pltpu.MemorySpace.VMEM),
                   pl.BlockSpec(memory_space=pltpu.MemorySpace.VMEM)),
        out_shape=(
            jax.ShapeDtypeStruct((_R // 2, _C), jnp.int32),
            jax.ShapeDtypeStruct((_R // 2, _C), jnp.float32),
        ),
        scratch_shapes=[
            pltpu.VMEM((HID, _KB), jnp.float32),
            pltpu.VMEM((HID, _KB), jnp.float32),
            pltpu.SemaphoreType.DMA((2, _NSPL)),
        ],
    )(tok2, x, w1, w2, b1, b2)


_R, _C = 8, 512


def _iotas():
    r = lax.broadcasted_iota(jnp.int32, (_R, _C), 0)
    c = lax.broadcasted_iota(jnp.int32, (_R, _C), 1)
    return r, c


def _roll(x, shift, axis):
    return pltpu.roll(x, shift % x.shape[axis], axis)


def _partner(x, j):
    if j < _C:
        lower = (_iotas()[1] & j) == 0
        return jnp.where(lower, _roll(x, -j, 1), _roll(x, j, 1))
    m = j // _C
    lower = (_iotas()[0] & m) == 0
    return jnp.where(lower, _roll(x, -m, 0), _roll(x, m, 0))


def _bitonic(key, val, descending, tiebreak=False):
    r, c = _iotas()
    k = 2
    while k <= _R * _C:
        j = k // 2
        while j >= 1:
            pk = _partner(key, j)
            pv = _partner(val, j)
            lower = ((c & j) == 0) if j < _C else ((r & (j // _C)) == 0)
            if k < _C:
                asc = (c & k) == 0
            elif k // _C < _R:
                asc = (r & (k // _C)) == 0
            else:
                asc = jnp.full((_R, _C), not descending)
            if k // _C < _R and descending:
                asc = jnp.logical_not(asc)
            keep_small = lower == asc
            if tiebreak:
                tie = pk == key
                p_lt = (pk < key) | (tie & (pv > val))
                p_gt = (pk > key) | (tie & (pv < val))
                swap = (keep_small & p_lt) | (~keep_small & p_gt)
            else:
                swap = (keep_small & (pk < key)) | (~keep_small & (pk > key))
            key = jnp.where(swap, pk, key)
            val = jnp.where(swap, pv, val)
            j //= 2
        k *= 2
    return key, val


def _bitonic_key(key, descending):
    r, c = _iotas()
    k = 2
    while k <= _R * _C:
        j = k // 2
        while j >= 1:
            pk = _partner(key, j)
            lower = ((c & j) == 0) if j < _C else ((r & (j // _C)) == 0)
            if k < _C:
                asc = (c & k) == 0
            elif k // _C < _R:
                asc = (r & (k // _C)) == 0
            else:
                asc = jnp.full((_R, _C), not descending)
            if k // _C < _R and descending:
                asc = jnp.logical_not(asc)
            keep_small = lower == asc
            swap = (keep_small & (pk < key)) | (~keep_small & (pk > key))
            key = jnp.where(swap, pk, key)
            j //= 2
        k *= 2
    return key


def _dedup_keep(tok):
    r, c = _iotas()
    pos = r * _C + c
    skey1 = _bitonic_key(tok * N + pos, descending=False)
    stok = skey1 >> 12
    nxt1 = _roll(stok, -1, 1)
    nxt = jnp.where(c == _C - 1, _roll(nxt1, -1, 0), nxt1)
    rep = (stok != nxt) | ((r == _R - 1) & (c == _C - 1))
    spos = skey1 & (N - 1)
    _, rep_pos = _bitonic(spos, rep.astype(jnp.int32), descending=False)
    return (rep_pos != 0) & (tok != PAD)


def _topk(keep, tok, sc):
    key2 = jnp.where(keep, sc, jnp.float32(-1e20))
    val2 = jnp.where(keep, tok, PAD)
    return _bitonic(key2, val2, descending=True, tiebreak=True)


def kernel(input_tokens, memory_context, emb, W1, b1, W2, b2):
    tokens = jnp.concatenate([input_tokens, memory_context], axis=0)
    rows = _sc_gather(emb, tokens)
    x = rows.reshape(2, _K)
    tt, ts = _fused_call(tokens.reshape(_R, _C), x, W1, W2,
                         b1.reshape(1, HID), b2.reshape(1, MAX_LEN))
    return tt.reshape(MAX_MEM), ts.reshape(MAX_MEM)
